# baseline (speedup 1.0000x reference)
import functools

import jax
import jax.numpy as jnp
from jax import lax
from jax.experimental import pallas as pl
from jax.experimental.pallas import tpu as pltpu
from jax.experimental.pallas import tpu_sc as plsc

D = 128
HALF = 64
LANES = 16
N_TILES = 32
WT = 128
TROWS = 600


def _lookup_add(x3, qidx, table, B, T):
    KB = B // WT
    n_total = T * KB
    wpt = n_total // N_TILES

    mesh = plsc.VectorSubcoreMesh(core_axis_name="c", subcore_axis_name="s")

    @functools.partial(
        pl.kernel,
        out_type=jax.ShapeDtypeStruct((B, T, D), jnp.float32),
        mesh=mesh,
        compiler_params=pltpu.CompilerParams(needs_layout_passes=False),
        scratch_types=[
            pltpu.VMEM((TROWS, D), jnp.float32),
            pltpu.VMEM((WT,), jnp.int32),
            pltpu.VMEM((WT,), jnp.int32),
            pltpu.VMEM((WT,), jnp.int32),
            pltpu.VMEM((WT,), jnp.int32),
            pltpu.VMEM((WT, D), jnp.float32),
            pltpu.VMEM((WT, D), jnp.float32),
            pltpu.SemaphoreType.DMA((2,)),
            pltpu.SemaphoreType.DMA((2,)),
            pltpu.SemaphoreType.DMA((2,)),
            pltpu.SemaphoreType.DMA((2,)),
        ],
    )
    def k(x_hbm, idx_hbm, tab_hbm, out_hbm,
          tab_v, ir0, ir1, ic0, ic1, xv0, xv1,
          irsem, icsem, xsem, osem):
        wid = lax.axis_index("s") * 2 + lax.axis_index("c")
        wbase = wid * wpt
        ir_b = (ir0, ir1)
        ic_b = (ic0, ic1)
        x_b = (xv0, xv1)

        def ir_copy(w, b):
            g = wbase + w
            return pltpu.make_async_copy(
                idx_hbm.at[pl.ds(g * 2 * WT, WT)], ir_b[b], irsem.at[b])

        def ic_copy(w, b):
            g = wbase + w
            return pltpu.make_async_copy(
                idx_hbm.at[pl.ds(g * 2 * WT + WT, WT)], ic_b[b], icsem.at[b])

        def idx_start(w, b):
            ir_copy(w, b).start()
            ic_copy(w, b).start()

        def idx_wait(w, b):
            ir_copy(w, b).wait()
            ic_copy(w, b).wait()

        def x_slice(w):
            g = wbase + w
            t = g // KB
            kk = g - t * KB
            return (pl.ds(kk * WT, WT), t)

        def x_copy(w, b):
            return pltpu.make_async_copy(x_hbm.at[x_slice(w)], x_b[b], xsem.at[b])

        def out_copy(w, b):
            return pltpu.make_async_copy(x_b[b], out_hbm.at[x_slice(w)], osem.at[b])

        idx_start(0, 0)
        idx_start(1, 1)
        x_copy(0, 0).start()
        pltpu.sync_copy(tab_hbm, tab_v)
        idx_wait(0, 0)

        @pl.loop(0, wpt // 2)
        def _(h):
            for b in (0, 1):
                w = 2 * h + b
                nb = 1 - b

                @pl.when(w >= 1)
                def _():
                    out_copy(w - 1, nb).wait()

                @pl.when(w + 1 < wpt)
                def _():
                    idx_wait(w + 1, nb)
                    x_copy(w + 1, nb).start()

                x_copy(w, b).wait()

                for g in range(WT // LANES):
                    gs = pl.ds(g * LANES, LANES)
                    r = ir_b[b][gs]
                    c = ic_b[b][gs]
                    rrow = lax.shift_right_logical(r, 1)
                    rlb = lax.shift_left(lax.bitwise_and(r, 1), 6)
                    crow = lax.shift_right_logical(c, 1)
                    clb = lax.shift_left(lax.bitwise_and(c, 1), 6)
                    tok = lax.iota(jnp.int32, LANES) + g * LANES

                    @pl.loop(0, HALF, unroll=4)
                    def _(w_):
                        wv = jnp.full((LANES,), w_, jnp.int32)
                        vr = plsc.load_gather(tab_v, [rrow, rlb + wv])
                        plsc.addupdate_scatter(x_b[b], [tok, wv], vr)
                        vc = plsc.load_gather(tab_v, [crow, clb + wv])
                        plsc.addupdate_scatter(x_b[b], [tok, wv + HALF], vc)

                @pl.when(w + 2 < wpt)
                def _():
                    idx_start(w + 2, b)

                out_copy(w, b).start()

        out_copy(wpt - 1, 1).wait()

    return k(x3, qidx, table)


def kernel(x, positions, pe):
    B, T, _ = x.shape
    qidx = (positions.transpose(1, 0, 2)
            .reshape(T, B // 128, 128, 2)
            .transpose(0, 1, 3, 2)
            .reshape(-1))
    table = pe.reshape(TROWS, D)
    out = _lookup_add(x, qidx, table, B, T)
    return out


# speedup vs baseline: 4.1123x; 4.1123x over previous
import functools

import jax
import jax.numpy as jnp
from jax import lax
from jax.experimental import pallas as pl
from jax.experimental.pallas import tpu as pltpu
from jax.experimental.pallas import tpu_sc as plsc

D = 128
HALF = 64
LANES = 16
N_TILES = 32
WT = 128
TROWS = 600


def _lookup_add(x3, qidx, table, B, T):
    KB = B // WT
    n_total = T * KB
    wpt = n_total // N_TILES

    mesh = plsc.VectorSubcoreMesh(core_axis_name="c", subcore_axis_name="s")

    @functools.partial(
        pl.kernel,
        out_type=jax.ShapeDtypeStruct((B, T, D), jnp.float32),
        mesh=mesh,
        compiler_params=pltpu.CompilerParams(needs_layout_passes=False),
        scratch_types=[
            pltpu.VMEM((TROWS, D), jnp.float32),
            pltpu.VMEM((WT,), jnp.int32),
            pltpu.VMEM((WT,), jnp.int32),
            pltpu.VMEM((WT,), jnp.int32),
            pltpu.VMEM((WT,), jnp.int32),
            pltpu.VMEM((WT, D), jnp.float32),
            pltpu.VMEM((WT, D), jnp.float32),
            pltpu.SemaphoreType.DMA((2,)),
            pltpu.SemaphoreType.DMA((2,)),
            pltpu.SemaphoreType.DMA((2,)),
            pltpu.SemaphoreType.DMA((2,)),
        ],
    )
    def k(x_hbm, idx_hbm, tab_hbm, out_hbm,
          tab_v, ir0, ir1, ic0, ic1, xv0, xv1,
          irsem, icsem, xsem, osem):
        wid = lax.axis_index("s") * 2 + lax.axis_index("c")
        wbase = wid * wpt
        ir_b = (ir0, ir1)
        ic_b = (ic0, ic1)
        x_b = (xv0, xv1)

        def ir_copy(w, b):
            g = wbase + w
            return pltpu.make_async_copy(
                idx_hbm.at[pl.ds(g * 2 * WT, WT)], ir_b[b], irsem.at[b])

        def ic_copy(w, b):
            g = wbase + w
            return pltpu.make_async_copy(
                idx_hbm.at[pl.ds(g * 2 * WT + WT, WT)], ic_b[b], icsem.at[b])

        def idx_start(w, b):
            ir_copy(w, b).start()
            ic_copy(w, b).start()

        def idx_wait(w, b):
            ir_copy(w, b).wait()
            ic_copy(w, b).wait()

        def x_slice(w):
            g = wbase + w
            t = g // KB
            kk = g - t * KB
            return (pl.ds(kk * WT, WT), t)

        def x_copy(w, b):
            return pltpu.make_async_copy(x_hbm.at[x_slice(w)], x_b[b], xsem.at[b])

        def out_copy(w, b):
            return pltpu.make_async_copy(x_b[b], out_hbm.at[x_slice(w)], osem.at[b])

        idx_start(0, 0)
        idx_start(1, 1)
        x_copy(0, 0).start()
        pltpu.sync_copy(tab_hbm, tab_v)
        idx_wait(0, 0)

        @pl.loop(0, wpt // 2)
        def _(h):
            for b in (0, 1):
                w = 2 * h + b
                nb = 1 - b

                @pl.when(w >= 1)
                def _():
                    out_copy(w - 1, nb).wait()

                @pl.when(w + 1 < wpt)
                def _():
                    idx_wait(w + 1, nb)
                    x_copy(w + 1, nb).start()

                x_copy(w, b).wait()

                iota = lax.iota(jnp.int32, LANES)

                @pl.loop(0, WT, unroll=4)
                def _(t_):
                    tv = jnp.full((LANES,), t_, jnp.int32)
                    rv = plsc.load_gather(ir_b[b], [tv])
                    cv = plsc.load_gather(ic_b[b], [tv])
                    rrow = lax.shift_right_logical(rv, 1)
                    rln = lax.shift_left(lax.bitwise_and(rv, 1), 6) + iota
                    crow = lax.shift_right_logical(cv, 1)
                    cln = lax.shift_left(lax.bitwise_and(cv, 1), 6) + iota
                    for j in range(4):
                        vr = plsc.load_gather(tab_v, [rrow, rln + j * LANES])
                        plsc.addupdate(x_b[b].at[t_, pl.ds(j * LANES, LANES)], vr)
                    for j in range(4):
                        vc = plsc.load_gather(tab_v, [crow, cln + j * LANES])
                        plsc.addupdate(
                            x_b[b].at[t_, pl.ds(HALF + j * LANES, LANES)], vc)

                @pl.when(w + 2 < wpt)
                def _():
                    idx_start(w + 2, b)

                out_copy(w, b).start()

        out_copy(wpt - 1, 1).wait()

    return k(x3, qidx, table)


def kernel(x, positions, pe):
    B, T, _ = x.shape
    qidx = (positions.transpose(1, 0, 2)
            .reshape(T, B // 128, 128, 2)
            .transpose(0, 1, 3, 2)
            .reshape(-1))
    table = pe.reshape(TROWS, D)
    out = _lookup_add(x, qidx, table, B, T)
    return out


# speedup vs baseline: 5.6251x; 1.3679x over previous
import functools

import jax
import jax.numpy as jnp
from jax import lax
from jax.experimental import pallas as pl
from jax.experimental.pallas import tpu as pltpu
from jax.experimental.pallas import tpu_sc as plsc

D = 128
LANES = 16
N_TILES = 32
WT = 128


def _lookup_add(x3, qidx, pe2, B, T):
    KB = B // WT
    n_total = T * KB
    wpt = n_total // N_TILES

    mesh = plsc.VectorSubcoreMesh(core_axis_name="c", subcore_axis_name="s")

    @functools.partial(
        pl.kernel,
        out_type=jax.ShapeDtypeStruct((B, T, D), jnp.float32),
        mesh=mesh,
        scratch_types=[
            pltpu.VMEM((WT,), jnp.int32),
            pltpu.VMEM((WT,), jnp.int32),
            pltpu.VMEM((WT,), jnp.int32),
            pltpu.VMEM((WT,), jnp.int32),
            pltpu.VMEM((WT, D), jnp.float32),
            pltpu.VMEM((WT, D), jnp.float32),
            pltpu.VMEM((WT, D), jnp.float32),
            pltpu.VMEM((WT, D), jnp.float32),
            pltpu.VMEM((WT, D), jnp.float32),
            pltpu.VMEM((WT, D), jnp.float32),
            pltpu.SemaphoreType.DMA((2,)),
            pltpu.SemaphoreType.DMA((2,)),
            pltpu.SemaphoreType.DMA((2,)),
            pltpu.SemaphoreType.DMA((2,)),
            pltpu.SemaphoreType.DMA((2,)),
            pltpu.SemaphoreType.DMA((2,)),
        ],
    )
    def k(x_hbm, idx_hbm, pe_hbm, out_hbm,
          ir0, ir1, ic0, ic1, gr0, gr1, gc0, gc1, xv0, xv1,
          irsem, icsem, rsem, csem, xsem, osem):
        wid = lax.axis_index("s") * 2 + lax.axis_index("c")
        wbase = wid * wpt
        ir_b = (ir0, ir1)
        ic_b = (ic0, ic1)
        gr_b = (gr0, gr1)
        gc_b = (gc0, gc1)
        x_b = (xv0, xv1)

        def ir_copy(w, b):
            g = wbase + w
            return pltpu.make_async_copy(
                idx_hbm.at[pl.ds(g * 2 * WT, WT)], ir_b[b], irsem.at[b])

        def ic_copy(w, b):
            g = wbase + w
            return pltpu.make_async_copy(
                idx_hbm.at[pl.ds(g * 2 * WT + WT, WT)], ic_b[b], icsem.at[b])

        def idx_start(w, b):
            ir_copy(w, b).start()
            ic_copy(w, b).start()

        def idx_wait(w, b):
            ir_copy(w, b).wait()
            ic_copy(w, b).wait()

        def row_gather(w, b):
            del w
            return pltpu.make_async_copy(pe_hbm.at[ir_b[b]], gr_b[b], rsem.at[b])

        def col_gather(w, b):
            del w
            return pltpu.make_async_copy(pe_hbm.at[ic_b[b]], gc_b[b], csem.at[b])

        def x_slice(w):
            g = wbase + w
            t = g // KB
            kk = g - t * KB
            return (pl.ds(kk * WT, WT), t)

        def x_copy(w, b):
            return pltpu.make_async_copy(x_hbm.at[x_slice(w)], x_b[b], xsem.at[b])

        def out_copy(w, b):
            return pltpu.make_async_copy(x_b[b], out_hbm.at[x_slice(w)], osem.at[b])

        idx_start(0, 0)
        idx_start(1, 1)
        idx_wait(0, 0)
        row_gather(0, 0).start()
        col_gather(0, 0).start()
        x_copy(0, 0).start()

        @pl.loop(0, wpt // 2)
        def _(h):
            for b in (0, 1):
                w = 2 * h + b
                nb = 1 - b

                @pl.when(w >= 1)
                def _():
                    out_copy(w - 1, nb).wait()

                @pl.when(w + 1 < wpt)
                def _():
                    idx_wait(w + 1, nb)
                    row_gather(w + 1, nb).start()
                    col_gather(w + 1, nb).start()
                    x_copy(w + 1, nb).start()

                row_gather(w, b).wait()
                col_gather(w, b).wait()
                x_copy(w, b).wait()

                @pl.when(w + 2 < wpt)
                def _():
                    idx_start(w + 2, b)

                @pl.loop(0, WT, unroll=8)
                def _(t):
                    for j in range(4):
                        s = pl.ds(j * LANES, LANES)
                        plsc.addupdate(x_b[b].at[t, s], gr_b[b][t, s])
                    for j in range(4, 8):
                        s = pl.ds(j * LANES, LANES)
                        plsc.addupdate(x_b[b].at[t, s], gc_b[b][t, s])

                out_copy(w, b).start()

        out_copy(wpt - 1, 1).wait()

    return k(x3, qidx, pe2)


def kernel(x, positions, pe):
    B, T, _ = x.shape
    qidx = (positions.transpose(1, 0, 2)
            .reshape(T, B // 128, 128, 2)
            .transpose(0, 1, 3, 2)
            .reshape(-1))
    pe2 = jnp.concatenate([pe, pe], axis=1)
    out = _lookup_add(x, qidx, pe2, B, T)
    return out


# speedup vs baseline: 5.6442x; 1.0034x over previous
import functools

import jax
import jax.numpy as jnp
from jax import lax
from jax.experimental import pallas as pl
from jax.experimental.pallas import tpu as pltpu
from jax.experimental.pallas import tpu_sc as plsc

D = 128
LANES = 16
N_TILES = 32
WT = 128


def _lookup_add(x3, qidx, pe2, B, T):
    KB = B // WT
    n_total = T * KB
    wpt = n_total // N_TILES

    mesh = plsc.VectorSubcoreMesh(core_axis_name="c", subcore_axis_name="s")

    @functools.partial(
        pl.kernel,
        out_type=jax.ShapeDtypeStruct((B, T, D), jnp.float32),
        mesh=mesh,
        scratch_types=[
            pltpu.VMEM((WT,), jnp.int32),
            pltpu.VMEM((WT,), jnp.int32),
            pltpu.VMEM((WT,), jnp.int32),
            pltpu.VMEM((WT,), jnp.int32),
            pltpu.VMEM((WT, D), jnp.float32),
            pltpu.VMEM((WT, D), jnp.float32),
            pltpu.VMEM((WT, D), jnp.float32),
            pltpu.VMEM((WT, D), jnp.float32),
            pltpu.VMEM((WT, D), jnp.float32),
            pltpu.VMEM((WT, D), jnp.float32),
            pltpu.SemaphoreType.DMA((2,)),
            pltpu.SemaphoreType.DMA((2,)),
            pltpu.SemaphoreType.DMA((2,)),
            pltpu.SemaphoreType.DMA((2,)),
            pltpu.SemaphoreType.DMA((2,)),
            pltpu.SemaphoreType.DMA((2,)),
        ],
    )
    def k(x_hbm, idx_hbm, pe_hbm, out_hbm,
          ir0, ir1, ic0, ic1, gr0, gr1, gc0, gc1, xv0, xv1,
          irsem, icsem, rsem, csem, xsem, osem):
        wid = lax.axis_index("s") * 2 + lax.axis_index("c")
        wbase = wid * wpt
        ir_b = (ir0, ir1)
        ic_b = (ic0, ic1)
        gr_b = (gr0, gr1)
        gc_b = (gc0, gc1)
        x_b = (xv0, xv1)

        def ir_copy(w, b):
            g = wbase + w
            return pltpu.make_async_copy(
                idx_hbm.at[pl.ds(g * 2 * WT, WT)], ir_b[b], irsem.at[b])

        def ic_copy(w, b):
            g = wbase + w
            return pltpu.make_async_copy(
                idx_hbm.at[pl.ds(g * 2 * WT + WT, WT)], ic_b[b], icsem.at[b])

        def idx_start(w, b):
            ir_copy(w, b).start()
            ic_copy(w, b).start()

        def idx_wait(w, b):
            ir_copy(w, b).wait()
            ic_copy(w, b).wait()

        def row_gather(w, b):
            del w
            return pltpu.make_async_copy(pe_hbm.at[ir_b[b]], gr_b[b], rsem.at[b])

        def col_gather(w, b):
            del w
            return pltpu.make_async_copy(pe_hbm.at[ic_b[b]], gc_b[b], csem.at[b])

        def x_slice(w):
            g = wbase + w
            t = g // KB
            kk = g - t * KB
            return (pl.ds(kk * WT, WT), t)

        def x_copy(w, b):
            return pltpu.make_async_copy(x_hbm.at[x_slice(w)], x_b[b], xsem.at[b])

        def out_copy(w, b):
            return pltpu.make_async_copy(x_b[b], out_hbm.at[x_slice(w)], osem.at[b])

        idx_start(0, 0)
        idx_start(1, 1)
        idx_wait(0, 0)
        row_gather(0, 0).start()
        col_gather(0, 0).start()
        x_copy(0, 0).start()

        @pl.loop(0, wpt // 2)
        def _(h):
            for b in (0, 1):
                w = 2 * h + b
                nb = 1 - b

                @pl.when(w + 1 < wpt)
                def _():
                    idx_wait(w + 1, nb)
                    row_gather(w + 1, nb).start()
                    col_gather(w + 1, nb).start()

                @pl.when(w >= 1)
                def _():
                    out_copy(w - 1, nb).wait()

                @pl.when(w + 1 < wpt)
                def _():
                    x_copy(w + 1, nb).start()

                row_gather(w, b).wait()
                col_gather(w, b).wait()
                x_copy(w, b).wait()

                @pl.when(w + 2 < wpt)
                def _():
                    idx_start(w + 2, b)

                @pl.loop(0, WT, unroll=8)
                def _(t):
                    for j in range(4):
                        s = pl.ds(j * LANES, LANES)
                        plsc.addupdate(x_b[b].at[t, s], gr_b[b][t, s])
                    for j in range(4, 8):
                        s = pl.ds(j * LANES, LANES)
                        plsc.addupdate(x_b[b].at[t, s], gc_b[b][t, s])

                out_copy(w, b).start()

        out_copy(wpt - 1, 1).wait()

    return k(x3, qidx, pe2)


def kernel(x, positions, pe):
    B, T, _ = x.shape
    qidx = (positions.transpose(1, 0, 2)
            .reshape(T, B // 128, 128, 2)
            .transpose(0, 1, 3, 2)
            .reshape(-1))
    pe2 = jnp.concatenate([pe, pe], axis=1)
    out = _lookup_add(x, qidx, pe2, B, T)
    return out
